# Initial kernel scaffold; baseline (speedup 1.0000x reference)
import jax
import jax.numpy as jnp
from jax.experimental import pallas as pl


def kernel(x, w1, b1, w2, b2, wf1, bf1, wf2, bf2, wf3, bf3):
    raise NotImplementedError("write your pallas kernel here")


# speedup vs baseline: 7.3933x; 7.3933x over previous
import numpy as np
import jax
import jax.numpy as jnp
from jax.experimental import pallas as pl
from jax.experimental.pallas import tpu as pltpu


def _net_kernel(x_ref, w0a_ref, w0b_ref, w1a_ref, w1b_ref,
                w2a_ref, w2b_ref, w3a_ref, w3b_ref, b1_ref,
                g2_ref, b2_ref, ce1_ref, co1_ref, ce2_ref, co2_ref,
                wf1_ref, bf1_ref, wf2_ref, bf2_ref, wf3_ref, bf3_ref,
                out_ref, p2a_ref, p2b_ref):
    BB = x_ref.shape[0] // 15
    S = BB * 15 - 1

    x4 = x_ref[...].astype(jnp.bfloat16)
    a0 = x4[0:S, :]
    a1 = x4[1:S + 1, :]
    b1 = b1_ref[...]

    def phase(pa, wa_ref, pb, wb_ref):
        acc = (jnp.dot(pa, wa_ref[...], preferred_element_type=jnp.float32)
               + jnp.dot(pb, wb_ref[...], preferred_element_type=jnp.float32))
        return jnp.maximum(acc + b1, 0.0)

    c1_0 = phase(a0, w0a_ref, a1[:, 0:60], w0b_ref)
    c1_1 = phase(a0[:, 60:240], w1a_ref, a1[:, 0:120], w1b_ref)
    c1_2 = phase(a0[:, 120:240], w2a_ref, a1[:, 0:180], w2b_ref)
    c1_3 = phase(a0[:, 180:240], w3a_ref, a1, w3b_ref)
    rm_lo = jnp.maximum(c1_0, c1_1).astype(jnp.bfloat16)
    rm_hi = jnp.maximum(c1_2, c1_3).astype(jnp.bfloat16)

    ce1 = ce1_ref[...]
    co1 = co1_ref[...]
    p1_lo = jnp.maximum(
        jnp.dot(rm_lo, ce1, preferred_element_type=jnp.float32),
        jnp.dot(rm_lo, co1, preferred_element_type=jnp.float32)
    ).astype(jnp.bfloat16)
    p1_hi = jnp.maximum(
        jnp.dot(rm_hi, ce1, preferred_element_type=jnp.float32),
        jnp.dot(rm_hi, co1, preferred_element_type=jnp.float32)
    ).astype(jnp.bfloat16)

    n2 = BB * 15 - 3
    g2 = g2_ref[...]
    lo = lambda k: p1_lo[k:k + n2, :]
    hi = lambda k: p1_hi[k:k + n2, :]
    G = lambda d: g2[d * 168:(d + 1) * 168, :]

    def dotf(a, b):
        return jnp.dot(a, b, preferred_element_type=jnp.float32)

    c2e = (dotf(lo(0), G(0)) + dotf(hi(0), G(1)) + dotf(lo(1), G(2))
           + dotf(hi(1), G(3)) + dotf(lo(2), G(4)))
    c2o = (dotf(hi(0), G(0)) + dotf(lo(1), G(1)) + dotf(hi(1), G(2))
           + dotf(lo(2), G(3)) + dotf(hi(2), G(4)))
    b2 = b2_ref[...]
    rm2 = jnp.maximum(jnp.maximum(c2e + b2, 0.0),
                      jnp.maximum(c2o + b2, 0.0)).astype(jnp.bfloat16)
    p2 = jnp.maximum(
        jnp.dot(rm2, ce2_ref[...], preferred_element_type=jnp.float32),
        jnp.dot(rm2, co2_ref[...], preferred_element_type=jnp.float32))
    p2a_ref[0:n2, :] = p2[:, 0:128]
    p2b_ref[0:n2, 0:64] = p2[:, 128:192]
    p2a_ref[n2:, :] = jnp.zeros((p2a_ref.shape[0] - n2, 128), jnp.float32)
    p2b_ref[n2:, 0:64] = jnp.zeros((p2b_ref.shape[0] - n2, 64), jnp.float32)

    wf1 = wf1_ref[...]

    def p2_rows(y):
        return jnp.concatenate(
            [p2a_ref[pl.dslice(y, BB, 15), :],
             p2b_ref[pl.dslice(y, BB, 15), 0:64]], axis=1).astype(jnp.bfloat16)

    h = jnp.dot(p2_rows(0), wf1[0:192, :], preferred_element_type=jnp.float32)
    for y in range(1, 12):
        h = h + jnp.dot(p2_rows(y), wf1[y * 192:(y + 1) * 192, :],
                        preferred_element_type=jnp.float32)
    h = jnp.maximum(h + bf1_ref[...], 0.0).astype(jnp.bfloat16)
    h = jnp.maximum(
        jnp.dot(h, wf2_ref[...], preferred_element_type=jnp.float32)
        + bf2_ref[...], 0.0).astype(jnp.bfloat16)
    out_ref[...] = (jnp.dot(h, wf3_ref[...], preferred_element_type=jnp.float32)
                    + bf3_ref[...])


def _absorbed_operators(w1, w2):
    s1 = np.stack([np.eye(60, 56, k=-d) for d in range(5)]).astype(np.float32)
    t1 = jnp.einsum('dpj,cyd->ypcj', s1, w1[:, 0]).reshape(300, 336)
    s2 = np.stack([np.eye(28, 24, k=-d) for d in range(5)]).astype(np.float32)
    g2 = jnp.einsum('dmx,ocyd->ycmox', s2, w2).reshape(840, 384)
    return t1, g2


def _col_selectors():
    c = np.repeat(np.arange(6), 28)
    k = np.tile(np.arange(28), 6)
    ce1 = np.zeros((336, 168), np.float32)
    co1 = np.zeros((336, 168), np.float32)
    ce1[c * 56 + 2 * k, c * 28 + k] = 1.0
    co1[c * 56 + 2 * k + 1, c * 28 + k] = 1.0
    o = np.repeat(np.arange(16), 12)
    t = np.tile(np.arange(12), 16)
    ce2 = np.zeros((384, 192), np.float32)
    co2 = np.zeros((384, 192), np.float32)
    ce2[o * 24 + 2 * t, o * 12 + t] = 1.0
    co2[o * 24 + 2 * t + 1, o * 12 + t] = 1.0
    return tuple(jnp.asarray(a) for a in (ce1, co1, ce2, co2))


def _pick_block(B, candidates):
    for c in candidates:
        if B % c == 0:
            return c
    return 1


def kernel(x, w1, b1, w2, b2, wf1, bf1, wf2, bf2, wf3, bf3):
    B = x.shape[0]
    BB = _pick_block(B, (64, 32, 16, 8, 4, 2))
    nb = B // BB

    t1, g2 = _absorbed_operators(w1, w2)
    t1 = t1.astype(jnp.bfloat16)
    phase_w = []
    for j in range(4):
        cut = (4 - j) * 60
        phase_w += [t1[0:cut, :], t1[cut:300, :]]
    b1_row = jnp.repeat(b1, 56)[None, :]
    b2_row = jnp.repeat(b2, 24)[None, :]
    ce1, co1, ce2, co2 = _col_selectors()
    wf1_perm = wf1.reshape(16, 12, 12, 120).transpose(1, 0, 2, 3)
    wf1_perm = wf1_perm.reshape(2304, 120)

    full = lambda a: pl.BlockSpec(a.shape, lambda b: tuple(0 for _ in a.shape))
    return pl.pallas_call(
        _net_kernel,
        grid=(nb,),
        in_specs=[pl.BlockSpec((BB * 15, 240), lambda b: (b, 0))]
        + [full(w) for w in phase_w]
        + [
            pl.BlockSpec((1, 336), lambda b: (0, 0)),
            full(g2), pl.BlockSpec((1, 384), lambda b: (0, 0)),
            full(ce1), full(co1), full(ce2), full(co2),
            full(wf1_perm), pl.BlockSpec((1, 120), lambda b: (0, 0)),
            full(wf2), pl.BlockSpec((1, 84), lambda b: (0, 0)),
            full(wf3), pl.BlockSpec((1, 2), lambda b: (0, 0)),
        ],
        out_specs=pl.BlockSpec((BB, 2), lambda b: (b, 0)),
        out_shape=jax.ShapeDtypeStruct((B, 2), jnp.float32),
        scratch_shapes=[
            pltpu.VMEM((BB * 15, 128), jnp.float32),
            pltpu.VMEM((BB * 15, 128), jnp.float32),
        ],
        compiler_params=pltpu.CompilerParams(
            dimension_semantics=("parallel",)),
    )(x.reshape(B * 15, 240), *phase_w, b1_row,
      g2.astype(jnp.bfloat16), b2_row,
      ce1.astype(jnp.bfloat16), co1.astype(jnp.bfloat16),
      ce2.astype(jnp.bfloat16), co2.astype(jnp.bfloat16),
      wf1_perm.astype(jnp.bfloat16), bf1.reshape(1, 120),
      wf2.astype(jnp.bfloat16), bf2.reshape(1, 84),
      wf3.astype(jnp.bfloat16), bf3.reshape(1, 2))
